# baseline (speedup 1.0000x reference)
import jax
import jax.numpy as jnp
from jax.experimental import pallas as pl
from jax.experimental.pallas import tpu as pltpu

DIM = 2048
HID = 4 * DIM
N_TOK = 2048
THRESH = 0.35
EPS = 1e-6
BH = 256
NJ = HID // BH


TT = 256
NT = N_TOK // TT


def _fused_block_kernel(x_ref, nw_ref, rw_ref, w1_ref, w2_ref, w3_ref,
                        gamma_ref, out_ref, acc_ref):
    j = pl.program_id(0)

    @pl.when(j == 0)
    def _prologue():
        for ti in range(NT):
            sl = pl.ds(ti * TT, TT)
            xf = x_ref[sl, :]
            ms = jnp.mean(xf * xf, axis=-1, keepdims=True)
            xn = xf * jax.lax.rsqrt(ms + EPS) * nw_ref[...]
            g = jnp.sum(xn * rw_ref[...], axis=-1, keepdims=True)
            act = (jax.nn.sigmoid(g) > THRESH).astype(jnp.float32)
            out_ref[sl, :] = xn * act

    for ti in range(NT):
        sl = pl.ds(ti * TT, TT)
        xt = out_ref[sl, :]
        u = jax.lax.dot_general(xt, w1_ref[...], (((1,), (1,)), ((), ())),
                                preferred_element_type=jnp.float32)
        v = jax.lax.dot_general(xt, w3_ref[...], (((1,), (1,)), ((), ())),
                                preferred_element_type=jnp.float32)
        h = u * jax.nn.sigmoid(u) * v
        t = jax.lax.dot_general(h, w2_ref[...], (((1,), (1,)), ((), ())),
                                preferred_element_type=jnp.float32)
        tb = t.astype(jnp.bfloat16)

        @pl.when(j == 0)
        def _init_acc():
            acc_ref[sl, :] = tb

        @pl.when(j > 0)
        def _accum():
            acc_ref[sl, :] += tb

    @pl.when(j == NJ - 1)
    def _epilogue():
        for ti in range(NT):
            sl = pl.ds(ti * TT, TT)
            out_ref[sl, :] = (x_ref[sl, :]
                              + acc_ref[sl, :].astype(jnp.float32) * gamma_ref[...])


@jax.jit
def kernel(x, norm_w, router_w, w1, w2, w3, gamma):
    nw = norm_w.reshape(1, DIM)
    gm = gamma.reshape(1, DIM)
    out = pl.pallas_call(
        _fused_block_kernel,
        grid=(NJ,),
        in_specs=[
            pl.BlockSpec((N_TOK, DIM), lambda j: (0, 0)),
            pl.BlockSpec((1, DIM), lambda j: (0, 0)),
            pl.BlockSpec((1, DIM), lambda j: (0, 0)),
            pl.BlockSpec((BH, DIM), lambda j: (j, 0)),
            pl.BlockSpec((DIM, BH), lambda j: (0, j)),
            pl.BlockSpec((BH, DIM), lambda j: (j, 0)),
            pl.BlockSpec((1, DIM), lambda j: (0, 0)),
        ],
        out_specs=pl.BlockSpec((N_TOK, DIM), lambda j: (0, 0)),
        out_shape=jax.ShapeDtypeStruct((N_TOK, DIM), jnp.float32),
        scratch_shapes=[
            pltpu.VMEM((N_TOK, DIM), jnp.bfloat16),
        ],
        compiler_params=pltpu.CompilerParams(
            vmem_limit_bytes=128 * 1024 * 1024,
        ),
    )(x, nw, router_w, w1, w2, w3, gm)
    return out


# speedup vs baseline: 1.1909x; 1.1909x over previous
import jax
import jax.numpy as jnp
from jax.experimental import pallas as pl
from jax.experimental.pallas import tpu as pltpu

DIM = 2048
HID = 4 * DIM
N_TOK = 2048
THRESH = 0.35
EPS = 1e-6
BH = 512
NJ = HID // BH
TT = 256
NT = N_TOK // TT


def _norm_router_kernel(x_ref, nw_ref, rw_ref, xn_ref):
    xf = x_ref[...]
    ms = jnp.mean(xf * xf, axis=-1, keepdims=True)
    xn = xf * jax.lax.rsqrt(ms + EPS) * nw_ref[...]
    g = jnp.sum(xn * rw_ref[...], axis=-1, keepdims=True)
    act = (jax.nn.sigmoid(g) > THRESH).astype(jnp.float32)
    xn_ref[...] = (xn * act).astype(jnp.bfloat16)


def _ffn_kernel(xn_ref, w1_ref, w2_ref, w3_ref, acc_ref):
    j = pl.program_id(0)
    w1b = w1_ref[...].astype(jnp.bfloat16)
    w3b = w3_ref[...].astype(jnp.bfloat16)
    w2b = w2_ref[...].astype(jnp.bfloat16)
    for ti in range(NT):
        sl = pl.ds(ti * TT, TT)
        xt = xn_ref[sl, :]
        u = jax.lax.dot_general(xt, w1b, (((1,), (1,)), ((), ())),
                                preferred_element_type=jnp.float32)
        v = jax.lax.dot_general(xt, w3b, (((1,), (1,)), ((), ())),
                                preferred_element_type=jnp.float32)
        h = (u * jax.nn.sigmoid(u) * v).astype(jnp.bfloat16)
        t = jax.lax.dot_general(h, w2b, (((1,), (1,)), ((), ())),
                                preferred_element_type=jnp.float32)
        tb = t.astype(jnp.bfloat16)

        @pl.when(j == 0)
        def _init():
            acc_ref[sl, :] = tb

        @pl.when(j > 0)
        def _accum():
            acc_ref[sl, :] += tb


def _residual_kernel(x_ref, acc_ref, gamma_ref, out_ref):
    out_ref[...] = x_ref[...] + acc_ref[...].astype(jnp.float32) * gamma_ref[...]


@jax.jit
def kernel(x, norm_w, router_w, w1, w2, w3, gamma):
    nw = norm_w.reshape(1, DIM)
    gm = gamma.reshape(1, DIM)

    xn = pl.pallas_call(
        _norm_router_kernel,
        grid=(NT,),
        in_specs=[
            pl.BlockSpec((TT, DIM), lambda i: (i, 0)),
            pl.BlockSpec((1, DIM), lambda i: (0, 0)),
            pl.BlockSpec((1, DIM), lambda i: (0, 0)),
        ],
        out_specs=pl.BlockSpec((TT, DIM), lambda i: (i, 0)),
        out_shape=jax.ShapeDtypeStruct((N_TOK, DIM), jnp.bfloat16),
    )(x, nw, router_w)

    acc = pl.pallas_call(
        _ffn_kernel,
        grid=(NJ,),
        in_specs=[
            pl.BlockSpec((N_TOK, DIM), lambda j: (0, 0)),
            pl.BlockSpec((BH, DIM), lambda j: (j, 0)),
            pl.BlockSpec((DIM, BH), lambda j: (0, j)),
            pl.BlockSpec((BH, DIM), lambda j: (j, 0)),
        ],
        out_specs=pl.BlockSpec((N_TOK, DIM), lambda j: (0, 0)),
        out_shape=jax.ShapeDtypeStruct((N_TOK, DIM), jnp.bfloat16),
        compiler_params=pltpu.CompilerParams(
            vmem_limit_bytes=128 * 1024 * 1024,
        ),
    )(xn, w1, w2, w3)

    out = pl.pallas_call(
        _residual_kernel,
        grid=(NT,),
        in_specs=[
            pl.BlockSpec((TT, DIM), lambda i: (i, 0)),
            pl.BlockSpec((TT, DIM), lambda i: (i, 0)),
            pl.BlockSpec((1, DIM), lambda i: (0, 0)),
        ],
        out_specs=pl.BlockSpec((TT, DIM), lambda i: (i, 0)),
        out_shape=jax.ShapeDtypeStruct((N_TOK, DIM), jnp.float32),
    )(x, acc, gm)
    return out


# speedup vs baseline: 1.2826x; 1.0770x over previous
import jax
import jax.numpy as jnp
from jax.experimental import pallas as pl
from jax.experimental.pallas import tpu as pltpu

DIM = 2048
HID = 4 * DIM
N_TOK = 2048
THRESH = 0.35
EPS = 1e-6
BH = 256
NJ = HID // BH
TT = 256
NT = N_TOK // TT


def _norm_router_kernel(x_ref, nw_ref, rw_ref, xn_ref):
    xf = x_ref[...]
    ms = jnp.mean(xf * xf, axis=-1, keepdims=True)
    xn = xf * jax.lax.rsqrt(ms + EPS) * nw_ref[...]
    g = jnp.sum(xn * rw_ref[...], axis=-1, keepdims=True)
    act = (jax.nn.sigmoid(g) > THRESH).astype(jnp.float32)
    xn_ref[...] = (xn * act).astype(jnp.bfloat16)


def _ffn_kernel(xn_ref, w1_ref, w2_ref, w3_ref, acc_ref):
    j = pl.program_id(0)
    w1b = w1_ref[...].astype(jnp.bfloat16)
    w3b = w3_ref[...].astype(jnp.bfloat16)
    w2b = w2_ref[...].astype(jnp.bfloat16)
    xt = xn_ref[...]
    u = jax.lax.dot_general(xt, w1b, (((1,), (1,)), ((), ())),
                            preferred_element_type=jnp.float32)
    v = jax.lax.dot_general(xt, w3b, (((1,), (1,)), ((), ())),
                            preferred_element_type=jnp.float32)
    h = (u * jax.nn.sigmoid(u) * v).astype(jnp.bfloat16)
    t = jax.lax.dot_general(h, w2b, (((1,), (1,)), ((), ())),
                            preferred_element_type=jnp.float32)
    tb = t.astype(jnp.bfloat16)

    @pl.when(j == 0)
    def _init():
        acc_ref[...] = tb

    @pl.when(j > 0)
    def _accum():
        acc_ref[...] += tb


def _residual_kernel(x_ref, acc_ref, gamma_ref, out_ref):
    out_ref[...] = x_ref[...] + acc_ref[...].astype(jnp.float32) * gamma_ref[...]


@jax.jit
def kernel(x, norm_w, router_w, w1, w2, w3, gamma):
    nw = norm_w.reshape(1, DIM)
    gm = gamma.reshape(1, DIM)

    xn = pl.pallas_call(
        _norm_router_kernel,
        grid=(NT,),
        in_specs=[
            pl.BlockSpec((TT, DIM), lambda i: (i, 0)),
            pl.BlockSpec((1, DIM), lambda i: (0, 0)),
            pl.BlockSpec((1, DIM), lambda i: (0, 0)),
        ],
        out_specs=pl.BlockSpec((TT, DIM), lambda i: (i, 0)),
        out_shape=jax.ShapeDtypeStruct((N_TOK, DIM), jnp.bfloat16),
    )(x, nw, router_w)

    acc = pl.pallas_call(
        _ffn_kernel,
        grid=(NJ,),
        in_specs=[
            pl.BlockSpec((N_TOK, DIM), lambda j: (0, 0)),
            pl.BlockSpec((BH, DIM), lambda j: (j, 0)),
            pl.BlockSpec((DIM, BH), lambda j: (0, j)),
            pl.BlockSpec((BH, DIM), lambda j: (j, 0)),
        ],
        out_specs=pl.BlockSpec((N_TOK, DIM), lambda j: (0, 0)),
        out_shape=jax.ShapeDtypeStruct((N_TOK, DIM), jnp.bfloat16),
        compiler_params=pltpu.CompilerParams(
            vmem_limit_bytes=128 * 1024 * 1024,
        ),
    )(xn, w1, w2, w3)

    out = pl.pallas_call(
        _residual_kernel,
        grid=(NT,),
        in_specs=[
            pl.BlockSpec((TT, DIM), lambda i: (i, 0)),
            pl.BlockSpec((TT, DIM), lambda i: (i, 0)),
            pl.BlockSpec((1, DIM), lambda i: (0, 0)),
        ],
        out_specs=pl.BlockSpec((TT, DIM), lambda i: (i, 0)),
        out_shape=jax.ShapeDtypeStruct((N_TOK, DIM), jnp.float32),
    )(x, acc, gm)
    return out


# speedup vs baseline: 1.3358x; 1.0414x over previous
import jax
import jax.numpy as jnp
from jax.experimental import pallas as pl
from jax.experimental.pallas import tpu as pltpu

DIM = 2048
HID = 4 * DIM
N_TOK = 2048
THRESH = 0.35
EPS = 1e-6
BH = 256
NJ = HID // BH
TT = 256
NT = N_TOK // TT
NSTEPS = NT + NJ + NT


def _fused_block_kernel(x_ref, nw_ref, rw_ref, w1_ref, w2_ref, w3_ref,
                        gamma_ref, out_ref, xn_ref, acc_ref):
    j = pl.program_id(0)

    @pl.when(j < NT)
    def _norm_phase():
        xf = x_ref[...]
        ms = jnp.mean(xf * xf, axis=-1, keepdims=True)
        xn = xf * jax.lax.rsqrt(ms + EPS) * nw_ref[...]
        g = jnp.sum(xn * rw_ref[...], axis=-1, keepdims=True)
        act = (jax.nn.sigmoid(g) > THRESH).astype(jnp.float32)
        xn_ref[pl.ds(j * TT, TT), :] = (xn * act).astype(jnp.bfloat16)

    @pl.when(jnp.logical_and(j >= NT, j < NT + NJ))
    def _ffn_phase():
        w1b = w1_ref[...].astype(jnp.bfloat16)
        w3b = w3_ref[...].astype(jnp.bfloat16)
        w2b = w2_ref[...].astype(jnp.bfloat16)
        xt = xn_ref[...]
        u = jax.lax.dot_general(xt, w1b, (((1,), (1,)), ((), ())),
                                preferred_element_type=jnp.float32)
        v = jax.lax.dot_general(xt, w3b, (((1,), (1,)), ((), ())),
                                preferred_element_type=jnp.float32)
        h = (u * jax.nn.sigmoid(u) * v).astype(jnp.bfloat16)
        t = jax.lax.dot_general(h, w2b, (((1,), (1,)), ((), ())),
                                preferred_element_type=jnp.float32)
        tb = t.astype(jnp.bfloat16)

        @pl.when(j == NT)
        def _init():
            acc_ref[...] = tb

        @pl.when(j > NT)
        def _accum():
            acc_ref[...] += tb

    @pl.when(j >= NT + NJ)
    def _epilogue_phase():
        ti = j - NT - NJ
        out_ref[...] = (x_ref[...]
                        + acc_ref[pl.ds(ti * TT, TT), :].astype(jnp.float32)
                        * gamma_ref[...])


def _x_idx(j):
    return (jnp.where(j < NT, j,
                      jnp.where(j < NT + NJ, NT - 1, j - NT - NJ)), 0)


def _w_row_idx(j):
    return (jnp.clip(j - NT, 0, NJ - 1), 0)


def _w_col_idx(j):
    return (0, jnp.clip(j - NT, 0, NJ - 1))


def _out_idx(j):
    return (jnp.maximum(j - NT - NJ, 0), 0)


@jax.jit
def kernel(x, norm_w, router_w, w1, w2, w3, gamma):
    nw = norm_w.reshape(1, DIM)
    gm = gamma.reshape(1, DIM)
    out = pl.pallas_call(
        _fused_block_kernel,
        grid=(NSTEPS,),
        in_specs=[
            pl.BlockSpec((TT, DIM), _x_idx),
            pl.BlockSpec((1, DIM), lambda j: (0, 0)),
            pl.BlockSpec((1, DIM), lambda j: (0, 0)),
            pl.BlockSpec((BH, DIM), _w_row_idx),
            pl.BlockSpec((DIM, BH), _w_col_idx),
            pl.BlockSpec((BH, DIM), _w_row_idx),
            pl.BlockSpec((1, DIM), lambda j: (0, 0)),
        ],
        out_specs=pl.BlockSpec((TT, DIM), _out_idx),
        out_shape=jax.ShapeDtypeStruct((N_TOK, DIM), jnp.float32),
        scratch_shapes=[
            pltpu.VMEM((N_TOK, DIM), jnp.bfloat16),
            pltpu.VMEM((N_TOK, DIM), jnp.bfloat16),
        ],
        compiler_params=pltpu.CompilerParams(
            vmem_limit_bytes=128 * 1024 * 1024,
        ),
    )(x, nw, router_w, w1, w2, w3, gm)
    return out
